# baseline (speedup 1.0000x reference)
import functools

import jax
import jax.numpy as jnp
from jax import lax
from jax.experimental import pallas as pl
from jax.experimental.pallas import tpu as pltpu
from jax.experimental.pallas import tpu_sc as plsc

NW = 32
NB1 = 4
S, B, D = 50, 16384, 32
V = 1000000
NT_FULL = 768
NROW = 800
NFB = 1536
NPB = 128


def _wid():
    return lax.axis_index("s") * 2 + lax.axis_index("c")


NLT = 7813
WJPW = 245


def _make_k1():
    mesh = plsc.VectorSubcoreMesh(core_axis_name="c", subcore_axis_name="s")

    @functools.partial(
        pl.kernel, mesh=mesh,
        out_type=(jax.ShapeDtypeStruct((NROW, 8, 128), jnp.int32),
                  jax.ShapeDtypeStruct((NLT, 32, 128), jnp.float32)),
        scratch_types=[
            pltpu.VMEM((NB1, 8, 128), jnp.int32),
            pltpu.VMEM((NB1, 32, 128), jnp.float32),
            pltpu.SemaphoreType.DMA((NB1,)),
            pltpu.SemaphoreType.DMA((NB1,)),
            pltpu.SemaphoreType.DMA((NB1,)),
            pltpu.SemaphoreType.DMA((NB1,)),
        ],
        compiler_params=pltpu.CompilerParams(use_tc_tiling_on_sc=True),
    )
    def k1(idx_hbm, wt_hbm, idx2_hbm, wtiles_hbm, vb, wb,
           isem, osem, wisem, wosem):
        wid = _wid()

        def w_src(lt):
            col = jnp.minimum(lt, NLT - 2) * 128
            return wt_hbm.at[:, pl.ds(pl.multiple_of(col, 128), 128)]

        def w_dst(lt):
            return wtiles_hbm.at[jnp.minimum(lt, NLT - 2)]

        def w_read(k, sl):
            lt = k * NW + wid
            pltpu.async_copy(w_src(lt), wb.at[sl], wisem.at[sl])

        def w_wait_read(k, sl):
            lt = k * NW + wid
            pltpu.make_async_copy(w_src(lt), wb.at[sl], wisem.at[sl]).wait()

        def w_write(k, sl):
            lt = k * NW + wid
            pltpu.async_copy(wb.at[sl], w_dst(lt), wosem.at[sl])

        def w_wait_write(k, sl):
            lt = k * NW + wid
            pltpu.make_async_copy(wb.at[sl], w_dst(lt), wosem.at[sl]).wait()

        pltpu.sync_copy(w_src(wid), wb.at[0])
        pltpu.async_copy(wb.at[0], w_dst(wid), wosem.at[0])
        w_wait_write(0, 0)
        for sl in range(NB1):
            w_read(1 + sl, sl)

        def w_quad(i, carry):
            for sl in range(NB1):
                k = 1 + 4 * i + sl
                w_wait_read(k, sl)
                w_write(k, sl)
            for sl in range(NB1):
                k = 1 + 4 * i + sl
                w_wait_write(k, sl)
                nk = k + NB1
                w_read(nk, sl)
            return carry

        lax.fori_loop(0, (WJPW - 1) // NB1 - 1, w_quad, 0)
        for sl in range(NB1):
            k = WJPW - NB1 + sl
            w_wait_read(k, sl)
            w_write(k, sl)
        for sl in range(NB1):
            k = WJPW - NB1 + sl
            w_wait_write(k, sl)


        def rd_src(j):
            g = j * NW + wid
            if j < 24:
                st, bt = g // 128, g % 128
                return idx_hbm.at[pl.ds(st * 8, 8), pl.ds(bt * 128, 128)]
            bt = g - NT_FULL
            return idx_hbm.at[pl.ds(48, 2), pl.ds(bt * 128, 128)]

        def wr_dst(j):
            g = j * NW + wid
            if j < 24:
                return idx2_hbm.at[g]
            bt = g - NT_FULL
            return idx2_hbm.at[NT_FULL + bt // 4, pl.ds((bt % 4) * 2, 2), :]

        def vb_ref(j):
            s = j % NB1
            return vb.at[s] if j < 24 else vb.at[s, pl.ds(0, 2), :]

        for j in range(NB1):
            pltpu.async_copy(rd_src(j), vb_ref(j), isem.at[j % NB1])
        for j in range(28):
            s = j % NB1
            pltpu.make_async_copy(rd_src(j), vb_ref(j), isem.at[s]).wait()
            pltpu.async_copy(vb_ref(j), wr_dst(j), osem.at[s])
            if j + NB1 < 28:
                pltpu.make_async_copy(vb_ref(j), wr_dst(j), osem.at[s]).wait()
                pltpu.async_copy(rd_src(j + NB1), vb_ref(j + NB1), isem.at[s])
        for j in range(24, 28):
            s = j % NB1
            pltpu.make_async_copy(vb_ref(j), wr_dst(j), osem.at[s]).wait()

    return k1


def _make_k2b():
    mesh = plsc.VectorSubcoreMesh(core_axis_name="c", subcore_axis_name="s")

    @functools.partial(
        pl.kernel, mesh=mesh,
        out_type=jax.ShapeDtypeStruct((V, D), jnp.float32),
        scratch_types=[
            pltpu.VMEM((2, 32, 128), jnp.float32),
            pltpu.VMEM((2, 128, 33), jnp.float32),
            pltpu.VMEM((64, D), jnp.float32),
            pltpu.SemaphoreType.DMA((2,)),
            pltpu.SemaphoreType.DMA((2,)),
        ],
        compiler_params=pltpu.CompilerParams(use_tc_tiling_on_sc=False,
                                             needs_layout_passes=False),
    )
    def k2b(wtiles_hbm, wtail_hbm, wlin_hbm, sb, tp, tbuf, rsem, wsem):
        wid = _wid()
        lane = lax.iota(jnp.int32, 16)

        def lt_of(k):
            return jnp.minimum(k * NW + wid, NLT - 2)

        def r0_of(k):
            return lt_of(k) * 128

        def start_read(k, sl):
            pltpu.async_copy(wtiles_hbm.at[lt_of(k)], sb.at[sl], rsem.at[sl])

        def wait_read(k, sl):
            pltpu.make_async_copy(wtiles_hbm.at[lt_of(k)], sb.at[sl],
                                  rsem.at[sl]).wait()

        def start_write(k, sl):
            pltpu.async_copy(tp.at[sl, :, pl.ds(0, 32)],
                             wlin_hbm.at[pl.ds(r0_of(k), 128), :],
                             wsem.at[sl])

        def wait_write(k, sl):
            pltpu.make_async_copy(tp.at[sl, :, pl.ds(0, 32)],
                                  wlin_hbm.at[pl.ds(r0_of(k), 128), :],
                                  wsem.at[sl]).wait()

        def transpose(sl):
            ssl, tsl = sb.at[sl], tp.at[sl]

            def body(c, carry):
                for l in range(8):
                    v = ssl[c, pl.ds(l * 16, 16)]
                    plsc.store_scatter(tsl, [l * 16 + lane,
                                             jnp.full((16,), c, jnp.int32)],
                                       v)
                return carry

            lax.fori_loop(0, 32, body, 0)

        start_read(0, 0)
        wait_read(0, 0)
        transpose(0)
        start_write(0, 0)
        start_read(1, 0)
        start_read(2, 1)

        def pair(i, carry):
            a = 1 + 2 * i
            wait_read(a, 0)
            wait_write(a - 1, 0)
            transpose(0)
            start_write(a, 0)
            start_read(a + 2, 0)
            wait_read(a + 1, 1)

            @pl.when(i > 0)
            def _():
                wait_write(a - 1, 1)
            transpose(1)
            start_write(a + 1, 1)
            start_read(a + 3, 1)
            return carry

        lax.fori_loop(0, (WJPW - 1) // 2 - 1, pair, 0)
        wait_read(243, 0)
        wait_write(242, 0)
        transpose(0)
        start_write(243, 0)
        wait_read(244, 1)
        wait_write(242, 1)
        transpose(1)
        start_write(244, 1)
        wait_write(243, 0)
        wait_write(244, 1)

        @pl.when(wid == 0)
        def _():
            pltpu.sync_copy(wtail_hbm, tbuf)
            pltpu.sync_copy(tbuf, wlin_hbm.at[pl.ds(V - 64, 64), :])

    return k2b


def _make_k2():
    mesh = plsc.VectorSubcoreMesh(core_axis_name="c", subcore_axis_name="s")

    @functools.partial(
        pl.kernel, mesh=mesh,
        out_type=jax.ShapeDtypeStruct((S, 4, 128, 8, 128), jnp.float32),
        scratch_types=[
            pltpu.VMEM((2, 4, 128), jnp.int32),
            pltpu.VMEM((2, 512, D), jnp.float32),
            pltpu.VMEM((2, 128, 129), jnp.float32),
            pltpu.SemaphoreType.DMA((2,)),
            pltpu.SemaphoreType.DMA((2,)),
            pltpu.SemaphoreType.DMA((2,)),
        ],
        compiler_params=pltpu.CompilerParams(use_tc_tiling_on_sc=False,
                                             needs_layout_passes=False),
    )
    def k2(idx2_hbm, w_hbm, out_hbm, ib, g, tb, isem, gsem, osem):
        wid = _wid()
        lane = lax.iota(jnp.int32, 16)

        def ib_src(k, nsub):
            hb = k * NW + wid
            if nsub == 4:
                hs, bt = hb // 128, hb % 128
                t = (hs // 2) * 128 + bt
                return idx2_hbm.at[t, pl.ds((hs % 2) * 4, 4), :]
            bt = hb
            return idx2_hbm.at[NT_FULL + bt // 4, pl.ds((bt % 4) * 2, 2), :]

        def ib_dst(sl, nsub):
            return ib.at[sl] if nsub == 4 else ib.at[sl, pl.ds(0, 2), :]

        def start_ib(k, sl, nsub):
            pltpu.async_copy(ib_src(k, nsub), ib_dst(sl, nsub), isem.at[sl])

        def wait_ib(k, sl, nsub):
            pltpu.make_async_copy(ib_src(k, nsub), ib_dst(sl, nsub),
                                  isem.at[sl]).wait()

        def start_gathers(sl, nsub):
            for su in range(nsub):
                pltpu.async_copy(w_hbm.at[ib.at[sl, su]],
                                 g.at[sl, pl.ds(su * 128, 128)], gsem.at[sl])

        def wait_gathers(sl, nsub):
            for su in range(nsub):
                pltpu.make_async_copy(w_hbm.at[ib.at[sl, su]],
                                      g.at[sl, pl.ds(su * 128, 128)],
                                      gsem.at[sl]).wait()

        def transpose(sl, nsub):
            gsl, tsl = g.at[sl], tb.at[sl]

            def body(b, carry):
                col = jnp.full((16,), b, jnp.int32)
                for su in range(nsub):
                    for c0 in (0, 16):
                        v = gsl[su * 128 + b, pl.ds(c0, 16)]
                        plsc.store_scatter(tsl, [su * 32 + c0 + lane, col], v)
                return carry

            lax.fori_loop(0, 128, body, 0)

        def start_outs(k, sl, nsub):
            hb = k * NW + wid
            if nsub == 4:
                hs, bt = hb // 128, hb % 128
                s0 = hs * 4
            else:
                s0, bt = 48, hb
            for su in range(nsub):
                for ct in range(4):
                    pltpu.async_copy(
                        tb.at[sl, pl.ds(su * 32 + ct * 8, 8), pl.ds(0, 128)],
                        out_hbm.at[s0 + su, ct, bt], osem.at[sl])

        def wait_outs(sl, nsub):
            for _ in range(4 * nsub):
                pltpu.make_async_copy(tb.at[sl, pl.ds(0, 8), pl.ds(0, 128)],
                                      out_hbm.at[0, 0, 0], osem.at[sl]).wait()

        nfull = NFB // NW

        def pair(i, first, last):
            a = 2 * i
            b = a + 1
            wait_ib(b, 1, 4)
            start_gathers(1, 4)
            wait_gathers(0, 4)
            if not last:
                start_ib(a + 2, 0, 4)
            if not first:
                wait_outs(0, 4)
            transpose(0, 4)
            start_outs(a, 0, 4)
            wait_gathers(1, 4)
            if not last:
                start_ib(b + 2, 1, 4)
            if not first:
                wait_outs(1, 4)
            transpose(1, 4)
            start_outs(b, 1, 4)
            if not last:
                wait_ib(a + 2, 0, 4)
                start_gathers(0, 4)
            return i

        start_ib(0, 0, 4)
        start_ib(1, 1, 4)
        wait_ib(0, 0, 4)
        start_gathers(0, 4)
        pair(0, True, False)
        lax.fori_loop(1, nfull // 2 - 1,
                      lambda i, c: pair(i, False, False), 0)
        pair(nfull // 2 - 1, False, True)
        wait_outs(0, 4)
        wait_outs(1, 4)

        for p in range(NPB // NW):
            pltpu.sync_copy(ib_src(p, 2), ib_dst(0, 2))
            start_gathers(0, 2)
            wait_gathers(0, 2)
            transpose(0, 2)
            start_outs(p, 0, 2)
            wait_outs(0, 2)

    return k2


_K1 = _make_k1()
_K2B = _make_k2b()
_K2 = _make_k2()


def kernel(token_ids, w):
    assert token_ids.shape == (B, S) and w.shape == (V, D)
    idx_t = token_ids.T
    w_t = w.T
    idx2, wtiles = _K1(idx_t, w_t)
    w_lin = _K2B(wtiles, w[V - 64:, :])
    out5 = _K2(idx2, w_lin)
    return out5.transpose(0, 1, 3, 2, 4).reshape(S, D, B).transpose(2, 0, 1)


# speedup vs baseline: 1.0550x; 1.0550x over previous
import functools

import jax
import jax.numpy as jnp
from jax import lax
from jax.experimental import pallas as pl
from jax.experimental.pallas import tpu as pltpu
from jax.experimental.pallas import tpu_sc as plsc

NW = 32
NB1 = 4
S, B, D = 50, 16384, 32
V = 1000000
NT_FULL = 768
NROW = 800
NFB = 1536
NPB = 128


def _wid():
    return lax.axis_index("s") * 2 + lax.axis_index("c")


NLT = 7813
WJPW = 245


def _make_k1():
    mesh = plsc.VectorSubcoreMesh(core_axis_name="c", subcore_axis_name="s")

    @functools.partial(
        pl.kernel, mesh=mesh,
        out_type=(jax.ShapeDtypeStruct((NROW, 8, 128), jnp.int32),
                  jax.ShapeDtypeStruct((NLT, 32, 128), jnp.float32)),
        scratch_types=[
            pltpu.VMEM((NB1, 8, 128), jnp.int32),
            pltpu.VMEM((NB1, 32, 128), jnp.float32),
            pltpu.SemaphoreType.DMA((NB1,)),
            pltpu.SemaphoreType.DMA((NB1,)),
            pltpu.SemaphoreType.DMA((NB1,)),
            pltpu.SemaphoreType.DMA((NB1,)),
        ],
        compiler_params=pltpu.CompilerParams(use_tc_tiling_on_sc=True),
    )
    def k1(idx_hbm, wt_hbm, idx2_hbm, wtiles_hbm, vb, wb,
           isem, osem, wisem, wosem):
        wid = _wid()

        def w_src(lt):
            col = jnp.minimum(lt, NLT - 2) * 128
            return wt_hbm.at[:, pl.ds(pl.multiple_of(col, 128), 128)]

        def w_dst(lt):
            return wtiles_hbm.at[jnp.minimum(lt, NLT - 2)]

        def w_read(k, sl):
            lt = k * NW + wid
            pltpu.async_copy(w_src(lt), wb.at[sl], wisem.at[sl])

        def w_wait_read(k, sl):
            lt = k * NW + wid
            pltpu.make_async_copy(w_src(lt), wb.at[sl], wisem.at[sl]).wait()

        def w_write(k, sl):
            lt = k * NW + wid
            pltpu.async_copy(wb.at[sl], w_dst(lt), wosem.at[sl])

        def w_wait_write(k, sl):
            lt = k * NW + wid
            pltpu.make_async_copy(wb.at[sl], w_dst(lt), wosem.at[sl]).wait()

        pltpu.sync_copy(w_src(wid), wb.at[0])
        pltpu.async_copy(wb.at[0], w_dst(wid), wosem.at[0])
        w_wait_write(0, 0)
        for sl in range(NB1):
            w_read(1 + sl, sl)

        def w_quad(i, carry):
            for sl in range(NB1):
                k = 1 + 4 * i + sl
                w_wait_read(k, sl)
                w_write(k, sl)
            for sl in range(NB1):
                k = 1 + 4 * i + sl
                w_wait_write(k, sl)
                nk = k + NB1
                w_read(nk, sl)
            return carry

        lax.fori_loop(0, (WJPW - 1) // NB1 - 1, w_quad, 0)
        for sl in range(NB1):
            k = WJPW - NB1 + sl
            w_wait_read(k, sl)
            w_write(k, sl)
        for sl in range(NB1):
            k = WJPW - NB1 + sl
            w_wait_write(k, sl)


        def rd_src(j):
            g = j * NW + wid
            if j < 24:
                st, bt = g // 128, g % 128
                return idx_hbm.at[pl.ds(st * 8, 8), pl.ds(bt * 128, 128)]
            bt = g - NT_FULL
            return idx_hbm.at[pl.ds(48, 2), pl.ds(bt * 128, 128)]

        def wr_dst(j):
            g = j * NW + wid
            if j < 24:
                return idx2_hbm.at[g]
            bt = g - NT_FULL
            return idx2_hbm.at[NT_FULL + bt // 4, pl.ds((bt % 4) * 2, 2), :]

        def vb_ref(j):
            s = j % NB1
            return vb.at[s] if j < 24 else vb.at[s, pl.ds(0, 2), :]

        for j in range(NB1):
            pltpu.async_copy(rd_src(j), vb_ref(j), isem.at[j % NB1])
        for j in range(28):
            s = j % NB1
            pltpu.make_async_copy(rd_src(j), vb_ref(j), isem.at[s]).wait()
            pltpu.async_copy(vb_ref(j), wr_dst(j), osem.at[s])
            if j + NB1 < 28:
                pltpu.make_async_copy(vb_ref(j), wr_dst(j), osem.at[s]).wait()
                pltpu.async_copy(rd_src(j + NB1), vb_ref(j + NB1), isem.at[s])
        for j in range(24, 28):
            s = j % NB1
            pltpu.make_async_copy(vb_ref(j), wr_dst(j), osem.at[s]).wait()

    return k1


def _make_k2b():
    mesh = plsc.VectorSubcoreMesh(core_axis_name="c", subcore_axis_name="s")

    @functools.partial(
        pl.kernel, mesh=mesh,
        out_type=jax.ShapeDtypeStruct((V, D), jnp.float32),
        scratch_types=[
            pltpu.VMEM((2, 32, 128), jnp.float32),
            pltpu.VMEM((2, 128, 33), jnp.float32),
            pltpu.VMEM_SHARED((16, 2, 128, 32), jnp.float32),
            pltpu.VMEM((64, D), jnp.float32),
            pltpu.SemaphoreType.DMA((2,)),
            pltpu.SemaphoreType.DMA((2,)),
        ],
        compiler_params=pltpu.CompilerParams(use_tc_tiling_on_sc=False,
                                             needs_layout_passes=False),
    )
    def k2b(wtiles_hbm, wtail_hbm, wlin_hbm, sb, tp, tc, tbuf, rsem, wsem):
        wid = _wid()
        lane = lax.iota(jnp.int32, 16)

        def lt_of(k):
            return jnp.minimum(k * NW + wid, NLT - 2)

        def r0_of(k):
            return lt_of(k) * 128

        def start_read(k, sl):
            pltpu.async_copy(wtiles_hbm.at[lt_of(k)], sb.at[sl], rsem.at[sl])

        def wait_read(k, sl):
            pltpu.make_async_copy(wtiles_hbm.at[lt_of(k)], sb.at[sl],
                                  rsem.at[sl]).wait()

        sid = lax.axis_index("s")

        def start_write(k, sl):
            pltpu.sync_copy(tp.at[sl, :, pl.ds(0, 32)], tc.at[sid, sl])
            pltpu.async_copy(tc.at[sid, sl],
                             wlin_hbm.at[pl.ds(r0_of(k), 128), :],
                             wsem.at[sl])

        def wait_write(k, sl):
            pltpu.make_async_copy(tc.at[sid, sl],
                                  wlin_hbm.at[pl.ds(r0_of(k), 128), :],
                                  wsem.at[sl]).wait()

        def transpose(sl):
            ssl, tsl = sb.at[sl], tp.at[sl]

            def body(c, carry):
                for l in range(8):
                    v = ssl[c, pl.ds(l * 16, 16)]
                    plsc.store_scatter(tsl, [l * 16 + lane,
                                             jnp.full((16,), c, jnp.int32)],
                                       v)
                return carry

            lax.fori_loop(0, 32, body, 0)

        start_read(0, 0)
        wait_read(0, 0)
        transpose(0)
        start_write(0, 0)
        start_read(1, 0)
        start_read(2, 1)

        def pair(i, carry):
            a = 1 + 2 * i
            wait_read(a, 0)
            wait_write(a - 1, 0)
            transpose(0)
            start_write(a, 0)
            start_read(a + 2, 0)
            wait_read(a + 1, 1)

            @pl.when(i > 0)
            def _():
                wait_write(a - 1, 1)
            transpose(1)
            start_write(a + 1, 1)
            start_read(a + 3, 1)
            return carry

        lax.fori_loop(0, (WJPW - 1) // 2 - 1, pair, 0)
        wait_read(243, 0)
        wait_write(242, 0)
        transpose(0)
        start_write(243, 0)
        wait_read(244, 1)
        wait_write(242, 1)
        transpose(1)
        start_write(244, 1)
        wait_write(243, 0)
        wait_write(244, 1)

        @pl.when(wid == 0)
        def _():
            pltpu.sync_copy(wtail_hbm, tbuf)
            pltpu.sync_copy(tbuf, wlin_hbm.at[pl.ds(V - 64, 64), :])

    return k2b


def _make_k2():
    mesh = plsc.VectorSubcoreMesh(core_axis_name="c", subcore_axis_name="s")

    @functools.partial(
        pl.kernel, mesh=mesh,
        out_type=jax.ShapeDtypeStruct((S, 4, 128, 8, 128), jnp.float32),
        scratch_types=[
            pltpu.VMEM((2, 4, 128), jnp.int32),
            pltpu.VMEM((2, 512, D), jnp.float32),
            pltpu.VMEM((2, 128, 129), jnp.float32),
            pltpu.SemaphoreType.DMA((2,)),
            pltpu.SemaphoreType.DMA((2,)),
            pltpu.SemaphoreType.DMA((2,)),
        ],
        compiler_params=pltpu.CompilerParams(use_tc_tiling_on_sc=False,
                                             needs_layout_passes=False),
    )
    def k2(idx2_hbm, w_hbm, out_hbm, ib, g, tb, isem, gsem, osem):
        wid = _wid()
        lane = lax.iota(jnp.int32, 16)

        def ib_src(k, nsub):
            hb = k * NW + wid
            if nsub == 4:
                hs, bt = hb // 128, hb % 128
                t = (hs // 2) * 128 + bt
                return idx2_hbm.at[t, pl.ds((hs % 2) * 4, 4), :]
            bt = hb
            return idx2_hbm.at[NT_FULL + bt // 4, pl.ds((bt % 4) * 2, 2), :]

        def ib_dst(sl, nsub):
            return ib.at[sl] if nsub == 4 else ib.at[sl, pl.ds(0, 2), :]

        def start_ib(k, sl, nsub):
            pltpu.async_copy(ib_src(k, nsub), ib_dst(sl, nsub), isem.at[sl])

        def wait_ib(k, sl, nsub):
            pltpu.make_async_copy(ib_src(k, nsub), ib_dst(sl, nsub),
                                  isem.at[sl]).wait()

        def start_gathers(sl, nsub):
            for su in range(nsub):
                pltpu.async_copy(w_hbm.at[ib.at[sl, su]],
                                 g.at[sl, pl.ds(su * 128, 128)], gsem.at[sl])

        def wait_gathers(sl, nsub):
            for su in range(nsub):
                pltpu.make_async_copy(w_hbm.at[ib.at[sl, su]],
                                      g.at[sl, pl.ds(su * 128, 128)],
                                      gsem.at[sl]).wait()

        def transpose(sl, nsub):
            gsl, tsl = g.at[sl], tb.at[sl]

            def body(b, carry):
                col = jnp.full((16,), b, jnp.int32)
                for su in range(nsub):
                    for c0 in (0, 16):
                        v = gsl[su * 128 + b, pl.ds(c0, 16)]
                        plsc.store_scatter(tsl, [su * 32 + c0 + lane, col], v)
                return carry

            lax.fori_loop(0, 128, body, 0)

        def start_outs(k, sl, nsub):
            hb = k * NW + wid
            if nsub == 4:
                hs, bt = hb // 128, hb % 128
                s0 = hs * 4
            else:
                s0, bt = 48, hb
            for su in range(nsub):
                for ct in range(4):
                    pltpu.async_copy(
                        tb.at[sl, pl.ds(su * 32 + ct * 8, 8), pl.ds(0, 128)],
                        out_hbm.at[s0 + su, ct, bt], osem.at[sl])

        def wait_outs(sl, nsub):
            for _ in range(4 * nsub):
                pltpu.make_async_copy(tb.at[sl, pl.ds(0, 8), pl.ds(0, 128)],
                                      out_hbm.at[0, 0, 0], osem.at[sl]).wait()

        nfull = NFB // NW

        def pair(i, first, last):
            a = 2 * i
            b = a + 1
            wait_ib(b, 1, 4)
            start_gathers(1, 4)
            wait_gathers(0, 4)
            if not last:
                start_ib(a + 2, 0, 4)
            if not first:
                wait_outs(0, 4)
            transpose(0, 4)
            start_outs(a, 0, 4)
            wait_gathers(1, 4)
            if not last:
                start_ib(b + 2, 1, 4)
            if not first:
                wait_outs(1, 4)
            transpose(1, 4)
            start_outs(b, 1, 4)
            if not last:
                wait_ib(a + 2, 0, 4)
                start_gathers(0, 4)
            return i

        start_ib(0, 0, 4)
        start_ib(1, 1, 4)
        wait_ib(0, 0, 4)
        start_gathers(0, 4)
        pair(0, True, False)
        lax.fori_loop(1, nfull // 2 - 1,
                      lambda i, c: pair(i, False, False), 0)
        pair(nfull // 2 - 1, False, True)
        wait_outs(0, 4)
        wait_outs(1, 4)

        for p in range(NPB // NW):
            pltpu.sync_copy(ib_src(p, 2), ib_dst(0, 2))
            start_gathers(0, 2)
            wait_gathers(0, 2)
            transpose(0, 2)
            start_outs(p, 0, 2)
            wait_outs(0, 2)

    return k2


_K1 = _make_k1()
_K2B = _make_k2b()
_K2 = _make_k2()


def kernel(token_ids, w):
    assert token_ids.shape == (B, S) and w.shape == (V, D)
    idx_t = token_ids.T
    w_t = w.T
    idx2, wtiles = _K1(idx_t, w_t)
    w_lin = _K2B(wtiles, w[V - 64:, :])
    out5 = _K2(idx2, w_lin)
    return out5.transpose(0, 1, 3, 2, 4).reshape(S, D, B).transpose(2, 0, 1)


# speedup vs baseline: 1.0640x; 1.0085x over previous
import functools

import jax
import jax.numpy as jnp
from jax import lax
from jax.experimental import pallas as pl
from jax.experimental.pallas import tpu as pltpu
from jax.experimental.pallas import tpu_sc as plsc

NW = 32
NB1 = 4
S, B, D = 50, 16384, 32
V = 1000000
NT_FULL = 768
NROW = 800
NFB = 1536
NPB = 128


def _wid():
    return lax.axis_index("s") * 2 + lax.axis_index("c")


NLT = 7813
WJPW = 245


def _make_k1():
    mesh = plsc.VectorSubcoreMesh(core_axis_name="c", subcore_axis_name="s")

    @functools.partial(
        pl.kernel, mesh=mesh,
        out_type=(jax.ShapeDtypeStruct((NROW, 8, 128), jnp.int32),
                  jax.ShapeDtypeStruct((NLT, 32, 128), jnp.float32)),
        scratch_types=[
            pltpu.VMEM((NB1, 8, 128), jnp.int32),
            pltpu.VMEM((NB1, 32, 128), jnp.float32),
            pltpu.SemaphoreType.DMA((NB1,)),
            pltpu.SemaphoreType.DMA((NB1,)),
            pltpu.SemaphoreType.DMA((NB1,)),
            pltpu.SemaphoreType.DMA((NB1,)),
        ],
        compiler_params=pltpu.CompilerParams(use_tc_tiling_on_sc=True),
    )
    def k1(idx_hbm, wt_hbm, idx2_hbm, wtiles_hbm, vb, wb,
           isem, osem, wisem, wosem):
        wid = _wid()

        def w_src(lt):
            col = jnp.minimum(lt, NLT - 2) * 128
            return wt_hbm.at[:, pl.ds(pl.multiple_of(col, 128), 128)]

        def w_dst(lt):
            return wtiles_hbm.at[jnp.minimum(lt, NLT - 2)]

        def w_read(k, sl):
            lt = k * NW + wid
            pltpu.async_copy(w_src(lt), wb.at[sl], wisem.at[sl])

        def w_wait_read(k, sl):
            lt = k * NW + wid
            pltpu.make_async_copy(w_src(lt), wb.at[sl], wisem.at[sl]).wait()

        def w_write(k, sl):
            lt = k * NW + wid
            pltpu.async_copy(wb.at[sl], w_dst(lt), wosem.at[sl])

        def w_wait_write(k, sl):
            lt = k * NW + wid
            pltpu.make_async_copy(wb.at[sl], w_dst(lt), wosem.at[sl]).wait()

        pltpu.sync_copy(w_src(wid), wb.at[0])
        pltpu.async_copy(wb.at[0], w_dst(wid), wosem.at[0])
        w_wait_write(0, 0)
        for sl in range(NB1):
            w_read(1 + sl, sl)

        def w_quad(i, carry):
            for sl in range(NB1):
                k = 1 + 4 * i + sl
                w_wait_read(k, sl)
                w_write(k, sl)
            for sl in range(NB1):
                k = 1 + 4 * i + sl
                w_wait_write(k, sl)
                nk = k + NB1
                w_read(nk, sl)
            return carry

        lax.fori_loop(0, (WJPW - 1) // NB1 - 1, w_quad, 0)
        for sl in range(NB1):
            k = WJPW - NB1 + sl
            w_wait_read(k, sl)
            w_write(k, sl)
        for sl in range(NB1):
            k = WJPW - NB1 + sl
            w_wait_write(k, sl)


        def rd_src(j):
            g = j * NW + wid
            if j < 24:
                st, bt = g // 128, g % 128
                return idx_hbm.at[pl.ds(st * 8, 8), pl.ds(bt * 128, 128)]
            bt = g - NT_FULL
            return idx_hbm.at[pl.ds(48, 2), pl.ds(bt * 128, 128)]

        def wr_dst(j):
            g = j * NW + wid
            if j < 24:
                return idx2_hbm.at[g]
            bt = g - NT_FULL
            return idx2_hbm.at[NT_FULL + bt // 4, pl.ds((bt % 4) * 2, 2), :]

        def vb_ref(j):
            s = j % NB1
            return vb.at[s] if j < 24 else vb.at[s, pl.ds(0, 2), :]

        for j in range(NB1):
            pltpu.async_copy(rd_src(j), vb_ref(j), isem.at[j % NB1])
        for j in range(28):
            s = j % NB1
            pltpu.make_async_copy(rd_src(j), vb_ref(j), isem.at[s]).wait()
            pltpu.async_copy(vb_ref(j), wr_dst(j), osem.at[s])
            if j + NB1 < 28:
                pltpu.make_async_copy(vb_ref(j), wr_dst(j), osem.at[s]).wait()
                pltpu.async_copy(rd_src(j + NB1), vb_ref(j + NB1), isem.at[s])
        for j in range(24, 28):
            s = j % NB1
            pltpu.make_async_copy(vb_ref(j), wr_dst(j), osem.at[s]).wait()

    return k1


def _make_k2b():
    mesh = plsc.VectorSubcoreMesh(core_axis_name="c", subcore_axis_name="s")

    @functools.partial(
        pl.kernel, mesh=mesh,
        out_type=jax.ShapeDtypeStruct((V, D), jnp.float32),
        scratch_types=[
            pltpu.VMEM((2, 32, 128), jnp.float32),
            pltpu.VMEM((2, 128, 33), jnp.float32),
            pltpu.VMEM_SHARED((16, 2, 128, 32), jnp.float32),
            pltpu.VMEM((64, D), jnp.float32),
            pltpu.SemaphoreType.DMA((2,)),
            pltpu.SemaphoreType.DMA((2,)),
        ],
        compiler_params=pltpu.CompilerParams(use_tc_tiling_on_sc=False,
                                             needs_layout_passes=False),
    )
    def k2b(wtiles_hbm, wtail_hbm, wlin_hbm, sb, tp, tc, tbuf, rsem, wsem):
        wid = _wid()
        lane = lax.iota(jnp.int32, 16)

        def lt_of(k):
            return jnp.minimum(k * NW + wid, NLT - 2)

        def r0_of(k):
            return lt_of(k) * 128

        def start_read(k, sl):
            pltpu.async_copy(wtiles_hbm.at[lt_of(k)], sb.at[sl], rsem.at[sl])

        def wait_read(k, sl):
            pltpu.make_async_copy(wtiles_hbm.at[lt_of(k)], sb.at[sl],
                                  rsem.at[sl]).wait()

        sid = lax.axis_index("s")

        def start_write(k, sl):
            pltpu.sync_copy(tp.at[sl, :, pl.ds(0, 32)], tc.at[sid, sl])
            pltpu.async_copy(tc.at[sid, sl],
                             wlin_hbm.at[pl.ds(r0_of(k), 128), :],
                             wsem.at[sl])

        def wait_write(k, sl):
            pltpu.make_async_copy(tc.at[sid, sl],
                                  wlin_hbm.at[pl.ds(r0_of(k), 128), :],
                                  wsem.at[sl]).wait()

        def transpose(sl):
            ssl, tsl = sb.at[sl], tp.at[sl]

            def body(cc, carry):
                for q in range(4):
                    c = 4 * cc + q
                    for l in range(8):
                        v = ssl[c, pl.ds(l * 16, 16)]
                        plsc.store_scatter(
                            tsl, [l * 16 + lane,
                                  jnp.full((16,), c, jnp.int32)], v)
                return carry

            lax.fori_loop(0, 8, body, 0)

        start_read(0, 0)
        wait_read(0, 0)
        transpose(0)
        start_write(0, 0)
        start_read(1, 0)
        start_read(2, 1)

        def pair(i, carry):
            a = 1 + 2 * i
            wait_read(a, 0)
            wait_write(a - 1, 0)
            transpose(0)
            start_write(a, 0)
            start_read(a + 2, 0)
            wait_read(a + 1, 1)

            @pl.when(i > 0)
            def _():
                wait_write(a - 1, 1)
            transpose(1)
            start_write(a + 1, 1)
            start_read(a + 3, 1)
            return carry

        lax.fori_loop(0, (WJPW - 1) // 2 - 1, pair, 0)
        wait_read(243, 0)
        wait_write(242, 0)
        transpose(0)
        start_write(243, 0)
        wait_read(244, 1)
        wait_write(242, 1)
        transpose(1)
        start_write(244, 1)
        wait_write(243, 0)
        wait_write(244, 1)

        @pl.when(wid == 0)
        def _():
            pltpu.sync_copy(wtail_hbm, tbuf)
            pltpu.sync_copy(tbuf, wlin_hbm.at[pl.ds(V - 64, 64), :])

    return k2b


def _make_k2():
    mesh = plsc.VectorSubcoreMesh(core_axis_name="c", subcore_axis_name="s")

    @functools.partial(
        pl.kernel, mesh=mesh,
        out_type=jax.ShapeDtypeStruct((S, 4, 128, 8, 128), jnp.float32),
        scratch_types=[
            pltpu.VMEM((2, 4, 128), jnp.int32),
            pltpu.VMEM((2, 512, D), jnp.float32),
            pltpu.VMEM((2, 128, 129), jnp.float32),
            pltpu.SemaphoreType.DMA((2,)),
            pltpu.SemaphoreType.DMA((2,)),
            pltpu.SemaphoreType.DMA((2,)),
        ],
        compiler_params=pltpu.CompilerParams(use_tc_tiling_on_sc=False,
                                             needs_layout_passes=False),
    )
    def k2(idx2_hbm, w_hbm, out_hbm, ib, g, tb, isem, gsem, osem):
        wid = _wid()
        lane = lax.iota(jnp.int32, 16)

        def ib_src(k, nsub):
            hb = k * NW + wid
            if nsub == 4:
                hs, bt = hb // 128, hb % 128
                t = (hs // 2) * 128 + bt
                return idx2_hbm.at[t, pl.ds((hs % 2) * 4, 4), :]
            bt = hb
            return idx2_hbm.at[NT_FULL + bt // 4, pl.ds((bt % 4) * 2, 2), :]

        def ib_dst(sl, nsub):
            return ib.at[sl] if nsub == 4 else ib.at[sl, pl.ds(0, 2), :]

        def start_ib(k, sl, nsub):
            pltpu.async_copy(ib_src(k, nsub), ib_dst(sl, nsub), isem.at[sl])

        def wait_ib(k, sl, nsub):
            pltpu.make_async_copy(ib_src(k, nsub), ib_dst(sl, nsub),
                                  isem.at[sl]).wait()

        def start_gathers(sl, nsub):
            for su in range(nsub):
                pltpu.async_copy(w_hbm.at[ib.at[sl, su]],
                                 g.at[sl, pl.ds(su * 128, 128)], gsem.at[sl])

        def wait_gathers(sl, nsub):
            for su in range(nsub):
                pltpu.make_async_copy(w_hbm.at[ib.at[sl, su]],
                                      g.at[sl, pl.ds(su * 128, 128)],
                                      gsem.at[sl]).wait()

        def transpose(sl, nsub):
            gsl, tsl = g.at[sl], tb.at[sl]

            def body(bb, carry):
                for q in range(4):
                    b = 4 * bb + q
                    col = jnp.full((16,), b, jnp.int32)
                    for su in range(nsub):
                        for c0 in (0, 16):
                            v = gsl[su * 128 + b, pl.ds(c0, 16)]
                            plsc.store_scatter(
                                tsl, [su * 32 + c0 + lane, col], v)
                return carry

            lax.fori_loop(0, 32, body, 0)

        def start_outs(k, sl, nsub):
            hb = k * NW + wid
            if nsub == 4:
                hs, bt = hb // 128, hb % 128
                s0 = hs * 4
            else:
                s0, bt = 48, hb
            for su in range(nsub):
                for ct in range(4):
                    pltpu.async_copy(
                        tb.at[sl, pl.ds(su * 32 + ct * 8, 8), pl.ds(0, 128)],
                        out_hbm.at[s0 + su, ct, bt], osem.at[sl])

        def wait_outs(sl, nsub):
            for _ in range(4 * nsub):
                pltpu.make_async_copy(tb.at[sl, pl.ds(0, 8), pl.ds(0, 128)],
                                      out_hbm.at[0, 0, 0], osem.at[sl]).wait()

        nfull = NFB // NW

        def pair(i, first, last):
            a = 2 * i
            b = a + 1
            wait_ib(b, 1, 4)
            start_gathers(1, 4)
            wait_gathers(0, 4)
            if not last:
                start_ib(a + 2, 0, 4)
            if not first:
                wait_outs(0, 4)
            transpose(0, 4)
            start_outs(a, 0, 4)
            wait_gathers(1, 4)
            if not last:
                start_ib(b + 2, 1, 4)
            if not first:
                wait_outs(1, 4)
            transpose(1, 4)
            start_outs(b, 1, 4)
            if not last:
                wait_ib(a + 2, 0, 4)
                start_gathers(0, 4)
            return i

        start_ib(0, 0, 4)
        start_ib(1, 1, 4)
        wait_ib(0, 0, 4)
        start_gathers(0, 4)
        pair(0, True, False)
        lax.fori_loop(1, nfull // 2 - 1,
                      lambda i, c: pair(i, False, False), 0)
        pair(nfull // 2 - 1, False, True)
        wait_outs(0, 4)
        wait_outs(1, 4)

        for p in range(NPB // NW):
            pltpu.sync_copy(ib_src(p, 2), ib_dst(0, 2))
            start_gathers(0, 2)
            wait_gathers(0, 2)
            transpose(0, 2)
            start_outs(p, 0, 2)
            wait_outs(0, 2)

    return k2


_K1 = _make_k1()
_K2B = _make_k2b()
_K2 = _make_k2()


def kernel(token_ids, w):
    assert token_ids.shape == (B, S) and w.shape == (V, D)
    idx_t = token_ids.T
    w_t = w.T
    idx2, wtiles = _K1(idx_t, w_t)
    w_lin = _K2B(wtiles, w[V - 64:, :])
    out5 = _K2(idx2, w_lin)
    return out5.transpose(0, 1, 3, 2, 4).reshape(S, D, B).transpose(2, 0, 1)


# speedup vs baseline: 1.0793x; 1.0144x over previous
import functools

import jax
import jax.numpy as jnp
from jax import lax
from jax.experimental import pallas as pl
from jax.experimental.pallas import tpu as pltpu
from jax.experimental.pallas import tpu_sc as plsc

NW = 32
NB1 = 4
S, B, D = 50, 16384, 32
V = 1000000
NT_FULL = 768
NROW = 800
NFB = 1536
NPB = 128


def _wid():
    return lax.axis_index("s") * 2 + lax.axis_index("c")


NLT = 7813
WJPW = 245


def _make_k1():
    mesh = plsc.VectorSubcoreMesh(core_axis_name="c", subcore_axis_name="s")

    @functools.partial(
        pl.kernel, mesh=mesh,
        out_type=(jax.ShapeDtypeStruct((NROW, 8, 128), jnp.int32),
                  jax.ShapeDtypeStruct((NLT, 32, 128), jnp.float32)),
        scratch_types=[
            pltpu.VMEM((NB1, 8, 128), jnp.int32),
            pltpu.VMEM((NB1, 32, 128), jnp.float32),
            pltpu.SemaphoreType.DMA((NB1,)),
            pltpu.SemaphoreType.DMA((NB1,)),
            pltpu.SemaphoreType.DMA((NB1,)),
            pltpu.SemaphoreType.DMA((NB1,)),
        ],
        compiler_params=pltpu.CompilerParams(use_tc_tiling_on_sc=True),
    )
    def k1(idx_hbm, wt_hbm, idx2_hbm, wtiles_hbm, vb, wb,
           isem, osem, wisem, wosem):
        wid = _wid()

        def w_src(lt):
            col = jnp.minimum(lt, NLT - 2) * 128
            return wt_hbm.at[:, pl.ds(pl.multiple_of(col, 128), 128)]

        def w_dst(lt):
            return wtiles_hbm.at[jnp.minimum(lt, NLT - 2)]

        def w_read(k, sl):
            lt = k * NW + wid
            pltpu.async_copy(w_src(lt), wb.at[sl], wisem.at[sl])

        def w_wait_read(k, sl):
            lt = k * NW + wid
            pltpu.make_async_copy(w_src(lt), wb.at[sl], wisem.at[sl]).wait()

        def w_write(k, sl):
            lt = k * NW + wid
            pltpu.async_copy(wb.at[sl], w_dst(lt), wosem.at[sl])

        def w_wait_write(k, sl):
            lt = k * NW + wid
            pltpu.make_async_copy(wb.at[sl], w_dst(lt), wosem.at[sl]).wait()

        pltpu.sync_copy(w_src(wid), wb.at[0])
        pltpu.async_copy(wb.at[0], w_dst(wid), wosem.at[0])
        w_wait_write(0, 0)
        for sl in range(NB1):
            w_read(1 + sl, sl)

        def w_quad(i, carry):
            for sl in range(NB1):
                k = 1 + 4 * i + sl
                w_wait_read(k, sl)
                w_write(k, sl)
            for sl in range(NB1):
                k = 1 + 4 * i + sl
                w_wait_write(k, sl)
                nk = k + NB1
                w_read(nk, sl)
            return carry

        lax.fori_loop(0, (WJPW - 1) // NB1 - 1, w_quad, 0)
        for sl in range(NB1):
            k = WJPW - NB1 + sl
            w_wait_read(k, sl)
            w_write(k, sl)
        for sl in range(NB1):
            k = WJPW - NB1 + sl
            w_wait_write(k, sl)


        def rd_src(j):
            g = j * NW + wid
            if j < 24:
                st, bt = g // 128, g % 128
                return idx_hbm.at[pl.ds(st * 8, 8), pl.ds(bt * 128, 128)]
            bt = g - NT_FULL
            return idx_hbm.at[pl.ds(48, 2), pl.ds(bt * 128, 128)]

        def wr_dst(j):
            g = j * NW + wid
            if j < 24:
                return idx2_hbm.at[g]
            bt = g - NT_FULL
            return idx2_hbm.at[NT_FULL + bt // 4, pl.ds((bt % 4) * 2, 2), :]

        def vb_ref(j):
            s = j % NB1
            return vb.at[s] if j < 24 else vb.at[s, pl.ds(0, 2), :]

        for j in range(NB1):
            pltpu.async_copy(rd_src(j), vb_ref(j), isem.at[j % NB1])
        for j in range(28):
            s = j % NB1
            pltpu.make_async_copy(rd_src(j), vb_ref(j), isem.at[s]).wait()
            pltpu.async_copy(vb_ref(j), wr_dst(j), osem.at[s])
            if j + NB1 < 28:
                pltpu.make_async_copy(vb_ref(j), wr_dst(j), osem.at[s]).wait()
                pltpu.async_copy(rd_src(j + NB1), vb_ref(j + NB1), isem.at[s])
        for j in range(24, 28):
            s = j % NB1
            pltpu.make_async_copy(vb_ref(j), wr_dst(j), osem.at[s]).wait()

    return k1


def _make_k2b():
    mesh = plsc.VectorSubcoreMesh(core_axis_name="c", subcore_axis_name="s")

    @functools.partial(
        pl.kernel, mesh=mesh,
        out_type=jax.ShapeDtypeStruct((V, D), jnp.float32),
        scratch_types=[
            pltpu.VMEM((2, 32, 128), jnp.float32),
            pltpu.VMEM((2, 128, 33), jnp.float32),
            pltpu.VMEM_SHARED((16, 2, 2, 128, 32), jnp.float32),
            pltpu.VMEM((64, D), jnp.float32),
            pltpu.SemaphoreType.DMA((2,)),
            pltpu.SemaphoreType.DMA((2,)),
            pltpu.SemaphoreType.DMA((2,)),
        ],
        compiler_params=pltpu.CompilerParams(use_tc_tiling_on_sc=False,
                                             needs_layout_passes=False),
    )
    def k2b(wtiles_hbm, wtail_hbm, wlin_hbm, sb, tp, tc, tbuf,
            rsem, usem, wsem):
        wid = _wid()
        lane = lax.iota(jnp.int32, 16)

        def lt_of(k):
            return jnp.minimum(k * NW + wid, NLT - 2)

        def r0_of(k):
            return lt_of(k) * 128

        def start_read(k, sl):
            pltpu.async_copy(wtiles_hbm.at[lt_of(k)], sb.at[sl], rsem.at[sl])

        def wait_read(k, sl):
            pltpu.make_async_copy(wtiles_hbm.at[lt_of(k)], sb.at[sl],
                                  rsem.at[sl]).wait()

        sid = lax.axis_index("s")

        def start_unpitch(sl, p):
            pltpu.async_copy(tp.at[sl, :, pl.ds(0, 32)], tc.at[sid, sl, p],
                             usem.at[sl])

        def wait_unpitch(sl):
            pltpu.make_async_copy(tp.at[sl, :, pl.ds(0, 32)],
                                  tc.at[sid, sl, 0], usem.at[sl]).wait()

        def start_write(k, sl, p):
            pltpu.async_copy(tc.at[sid, sl, p],
                             wlin_hbm.at[pl.ds(r0_of(k), 128), :],
                             wsem.at[sl])

        def wait_write(k, sl):
            pltpu.make_async_copy(tc.at[sid, sl, 0],
                                  wlin_hbm.at[pl.ds(r0_of(k), 128), :],
                                  wsem.at[sl]).wait()

        def transpose(sl):
            ssl, tsl = sb.at[sl], tp.at[sl]

            def body(cc, carry):
                for q in range(4):
                    c = 4 * cc + q
                    for l in range(8):
                        v = ssl[c, pl.ds(l * 16, 16)]
                        plsc.store_scatter(
                            tsl, [l * 16 + lane,
                                  jnp.full((16,), c, jnp.int32)], v)
                return carry

            lax.fori_loop(0, 8, body, 0)

        pltpu.sync_copy(wtiles_hbm.at[lt_of(0)], sb.at[1])
        transpose(1)
        start_unpitch(1, 1)
        start_read(1, 0)
        start_read(2, 1)

        def pair(i, carry):
            a = 1 + 2 * i
            b = a + 1
            p = lax.rem(i, 2)

            wait_read(a, 0)

            @pl.when(i > 0)
            def _():
                wait_unpitch(0)
                start_write(a - 2, 0, 1 - p)
            transpose(0)

            @pl.when(i > 1)
            def _():
                wait_write(a, 0)
            start_unpitch(0, p)

            @pl.when(i < (WJPW - 1) // 2 - 1)
            def _():
                start_read(a + 2, 0)

            wait_read(b, 1)
            wait_unpitch(1)
            start_write(b - 2, 1, 1 - p)
            transpose(1)

            @pl.when(i > 0)
            def _():
                wait_write(b, 1)
            start_unpitch(1, p)

            @pl.when(i < (WJPW - 1) // 2 - 1)
            def _():
                start_read(b + 2, 1)
            return carry

        lax.fori_loop(0, (WJPW - 1) // 2, pair, 0)
        wait_unpitch(0)
        start_write(WJPW - 2, 0, 1)
        wait_unpitch(1)
        start_write(WJPW - 1, 1, 1)
        for sl in range(2):
            wait_write(0, sl)
            wait_write(0, sl)

        @pl.when(wid == 0)
        def _():
            pltpu.sync_copy(wtail_hbm, tbuf)
            pltpu.sync_copy(tbuf, wlin_hbm.at[pl.ds(V - 64, 64), :])

    return k2b


def _make_k2():
    mesh = plsc.VectorSubcoreMesh(core_axis_name="c", subcore_axis_name="s")

    @functools.partial(
        pl.kernel, mesh=mesh,
        out_type=jax.ShapeDtypeStruct((S, 4, 128, 8, 128), jnp.float32),
        scratch_types=[
            pltpu.VMEM((2, 4, 128), jnp.int32),
            pltpu.VMEM((2, 512, D), jnp.float32),
            pltpu.VMEM((2, 128, 129), jnp.float32),
            pltpu.SemaphoreType.DMA((2,)),
            pltpu.SemaphoreType.DMA((2,)),
            pltpu.SemaphoreType.DMA((2,)),
        ],
        compiler_params=pltpu.CompilerParams(use_tc_tiling_on_sc=False,
                                             needs_layout_passes=False),
    )
    def k2(idx2_hbm, w_hbm, out_hbm, ib, g, tb, isem, gsem, osem):
        wid = _wid()
        lane = lax.iota(jnp.int32, 16)

        def ib_src(k, nsub):
            hb = k * NW + wid
            if nsub == 4:
                hs, bt = hb // 128, hb % 128
                t = (hs // 2) * 128 + bt
                return idx2_hbm.at[t, pl.ds((hs % 2) * 4, 4), :]
            bt = hb
            return idx2_hbm.at[NT_FULL + bt // 4, pl.ds((bt % 4) * 2, 2), :]

        def ib_dst(sl, nsub):
            return ib.at[sl] if nsub == 4 else ib.at[sl, pl.ds(0, 2), :]

        def start_ib(k, sl, nsub):
            pltpu.async_copy(ib_src(k, nsub), ib_dst(sl, nsub), isem.at[sl])

        def wait_ib(k, sl, nsub):
            pltpu.make_async_copy(ib_src(k, nsub), ib_dst(sl, nsub),
                                  isem.at[sl]).wait()

        def start_gathers(sl, nsub):
            for su in range(nsub):
                pltpu.async_copy(w_hbm.at[ib.at[sl, su]],
                                 g.at[sl, pl.ds(su * 128, 128)], gsem.at[sl])

        def wait_gathers(sl, nsub):
            for su in range(nsub):
                pltpu.make_async_copy(w_hbm.at[ib.at[sl, su]],
                                      g.at[sl, pl.ds(su * 128, 128)],
                                      gsem.at[sl]).wait()

        def transpose(sl, nsub):
            gsl, tsl = g.at[sl], tb.at[sl]

            def body(bb, carry):
                for q in range(4):
                    b = 4 * bb + q
                    col = jnp.full((16,), b, jnp.int32)
                    for su in range(nsub):
                        for c0 in (0, 16):
                            v = gsl[su * 128 + b, pl.ds(c0, 16)]
                            plsc.store_scatter(
                                tsl, [su * 32 + c0 + lane, col], v)
                return carry

            lax.fori_loop(0, 32, body, 0)

        def start_outs(k, sl, nsub):
            hb = k * NW + wid
            if nsub == 4:
                hs, bt = hb // 128, hb % 128
                s0 = hs * 4
            else:
                s0, bt = 48, hb
            for su in range(nsub):
                for ct in range(4):
                    pltpu.async_copy(
                        tb.at[sl, pl.ds(su * 32 + ct * 8, 8), pl.ds(0, 128)],
                        out_hbm.at[s0 + su, ct, bt], osem.at[sl])

        def wait_outs(sl, nsub):
            for _ in range(4 * nsub):
                pltpu.make_async_copy(tb.at[sl, pl.ds(0, 8), pl.ds(0, 128)],
                                      out_hbm.at[0, 0, 0], osem.at[sl]).wait()

        nfull = NFB // NW

        def pair(i, first, last):
            a = 2 * i
            b = a + 1
            wait_ib(b, 1, 4)
            start_gathers(1, 4)
            wait_gathers(0, 4)
            if not last:
                start_ib(a + 2, 0, 4)
            if not first:
                wait_outs(0, 4)
            transpose(0, 4)
            start_outs(a, 0, 4)
            wait_gathers(1, 4)
            if not last:
                start_ib(b + 2, 1, 4)
            if not first:
                wait_outs(1, 4)
            transpose(1, 4)
            start_outs(b, 1, 4)
            if not last:
                wait_ib(a + 2, 0, 4)
                start_gathers(0, 4)
            return i

        start_ib(0, 0, 4)
        start_ib(1, 1, 4)
        wait_ib(0, 0, 4)
        start_gathers(0, 4)
        pair(0, True, False)
        lax.fori_loop(1, nfull // 2 - 1,
                      lambda i, c: pair(i, False, False), 0)
        pair(nfull // 2 - 1, False, True)
        wait_outs(0, 4)
        wait_outs(1, 4)

        for p in range(NPB // NW):
            pltpu.sync_copy(ib_src(p, 2), ib_dst(0, 2))
            start_gathers(0, 2)
            wait_gathers(0, 2)
            transpose(0, 2)
            start_outs(p, 0, 2)
            wait_outs(0, 2)

    return k2


_K1 = _make_k1()
_K2B = _make_k2b()
_K2 = _make_k2()


def kernel(token_ids, w):
    assert token_ids.shape == (B, S) and w.shape == (V, D)
    idx_t = token_ids.T
    w_t = w.T
    idx2, wtiles = _K1(idx_t, w_t)
    w_lin = _K2B(wtiles, w[V - 64:, :])
    out5 = _K2(idx2, w_lin)
    return out5.transpose(0, 1, 3, 2, 4).reshape(S, D, B).transpose(2, 0, 1)
